# Initial kernel scaffold; baseline (speedup 1.0000x reference)
import jax
import jax.numpy as jnp
from jax.experimental import pallas as pl


def kernel(input_):
    raise NotImplementedError("write your pallas kernel here")


# speedup vs baseline: 2.8795x; 2.8795x over previous
import functools

import jax
import jax.numpy as jnp
import numpy as np
from jax import lax
from jax.experimental import pallas as pl
from jax.experimental.pallas import tpu as pltpu
from jax.experimental.pallas import tpu_sc as plsc

ROWS = 128
COLS = 32768
K = 64
LANES = 16
NV = COLS // LANES
HIST = 256
CAP = 8192
NC = 2
NS = 16
NW = NC * NS
ROWS_PER_W = ROWS // NW

ABS_MASK = np.int32(0x7FFFFFFF)


def _body(in_hbm, out_hbm, xrow, hist, av, ai, bv, bi):
    wid = lax.axis_index("s") * NC + lax.axis_index("c")
    iota = lax.iota(jnp.int32, LANES)
    ones16 = jnp.ones((LANES,), jnp.int32)
    zeros16 = jnp.zeros((LANES,), jnp.int32)

    def clear_hist():
        @pl.loop(0, HIST // LANES)
        def _(h):
            hist[pl.ds(h * LANES, LANES)] = zeros16

    def hist_at(b):
        return hist[pl.ds(b, LANES)][0]

    def scan_hist(needed):
        def cond(st):
            b, cum = st
            return (cum < needed) & (b > 0)

        def step(st):
            b, cum = st
            b2 = b - 1
            return b2, cum + hist_at(b2)

        bstar, cum = lax.while_loop(
            cond, step, (np.int32(HIST), np.int32(0)))
        return bstar, needed - (cum - hist_at(bstar))

    def do_row(row):
        pltpu.sync_copy(in_hbm.at[row], xrow)

        clear_hist()

        @pl.loop(0, NV)
        def _(v):
            key = xrow[pl.ds(v * LANES, LANES)] & ABS_MASK
            plsc.addupdate_scatter(hist, [key >> 23], ones16)

        bstar, needed = scan_hist(np.int32(K))

        @pl.loop(0, NV, init_carry=np.int32(0))
        def nA(v, off):
            raw = xrow[pl.ds(v * LANES, LANES)]
            key = raw & ABS_MASK
            b = key >> 23
            meq = b == bstar
            cnt = plsc.cumsum(meq.astype(jnp.int32))
            pos = jnp.minimum(off + cnt - 1, CAP - 1)
            plsc.store_scatter(av, [pos], raw, mask=meq)
            plsc.store_scatter(ai, [pos], v * LANES + iota, mask=meq)
            xrow[pl.ds(v * LANES, LANES)] = jnp.where(b > bstar, raw, 0)
            return jnp.minimum(off + jnp.max(cnt), CAP)

        def refine(sv, si, dv, di, n, needed, shift, nbits):
            bmask = np.int32((1 << nbits) - 1)
            trips = (n + LANES - 1) // LANES
            clear_hist()

            @pl.loop(0, trips)
            def _(v):
                m = (v * LANES + iota) < n
                key = sv[pl.ds(v * LANES, LANES)] & ABS_MASK
                plsc.addupdate_scatter(
                    hist, [(key >> shift) & bmask], ones16, mask=m)

            bstar, needed2 = scan_hist(needed)

            @pl.loop(0, trips, init_carry=np.int32(0))
            def n2(v, off):
                lanem = (v * LANES + iota) < n
                raw = sv[pl.ds(v * LANES, LANES)]
                idx = si[pl.ds(v * LANES, LANES)]
                b = ((raw & ABS_MASK) >> shift) & bmask
                mgt = lanem & (b > bstar)
                plsc.store_scatter(xrow, [idx], raw, mask=mgt)
                meq = lanem & (b == bstar)
                cnt = plsc.cumsum(meq.astype(jnp.int32))
                pos = jnp.minimum(off + cnt - 1, CAP - 1)
                plsc.store_scatter(dv, [pos], raw, mask=meq)
                plsc.store_scatter(di, [pos], idx, mask=meq)
                return jnp.minimum(off + jnp.max(cnt), CAP)

            return n2, needed2

        nB, needed = refine(av, ai, bv, bi, nA, needed, 15, 8)
        nA2, needed = refine(bv, bi, av, ai, nB, needed, 7, 8)
        nT, needed = refine(av, ai, bv, bi, nA2, needed, 0, 7)

        tie_trips = (jnp.minimum(nT, needed) + LANES - 1) // LANES

        @pl.loop(0, tie_trips)
        def _(v):
            posv = v * LANES + iota
            m = (posv < nT) & (posv < needed)
            raw = bv[pl.ds(v * LANES, LANES)]
            idx = bi[pl.ds(v * LANES, LANES)]
            plsc.store_scatter(xrow, [idx], raw, mask=m)

        pltpu.sync_copy(xrow, out_hbm.at[row])

    @pl.loop(0, ROWS_PER_W)
    def _(r):
        do_row(wid * ROWS_PER_W + r)


@jax.jit
def kernel(input_):
    mesh = plsc.VectorSubcoreMesh(
        core_axis_name="c", subcore_axis_name="s",
        num_cores=NC, num_subcores=NS)
    f = pl.kernel(
        _body,
        out_type=jax.ShapeDtypeStruct((ROWS, COLS), jnp.int32),
        mesh=mesh,
        scratch_types=[
            pltpu.VMEM((COLS,), jnp.int32),
            pltpu.VMEM((HIST + LANES,), jnp.int32),
            pltpu.VMEM((CAP,), jnp.int32),
            pltpu.VMEM((CAP,), jnp.int32),
            pltpu.VMEM((CAP,), jnp.int32),
            pltpu.VMEM((CAP,), jnp.int32),
        ],
        compiler_params=pltpu.CompilerParams(needs_layout_passes=False),
        name="topk_abs_mask_sc",
    )
    bits = lax.bitcast_convert_type(input_, jnp.int32)
    return lax.bitcast_convert_type(f(bits), jnp.float32)


# speedup vs baseline: 3.5215x; 1.2230x over previous
import functools

import jax
import jax.numpy as jnp
import numpy as np
from jax import lax
from jax.experimental import pallas as pl
from jax.experimental.pallas import tpu as pltpu
from jax.experimental.pallas import tpu_sc as plsc

ROWS = 128
COLS = 32768
K = 64
LANES = 16
NV = COLS // LANES
HIST = 256
CAP = 8192
CAPB = CAP + LANES
NC = 2
NS = 16
NW = NC * NS
ROWS_PER_W = ROWS // NW

ABS_MASK = np.int32(0x7FFFFFFF)


def _body(in_hbm, out_hbm, xrow, hist, av, ai, bv, bi):
    wid = lax.axis_index("s") * NC + lax.axis_index("c")
    iota = lax.iota(jnp.int32, LANES)
    ones16 = jnp.ones((LANES,), jnp.int32)
    zeros16 = jnp.zeros((LANES,), jnp.int32)

    def clear_hist():
        @pl.loop(0, HIST // LANES)
        def _(h):
            hist[pl.ds(h * LANES, LANES)] = zeros16

    def hist_at(b):
        return hist[pl.ds(b, LANES)][0]

    def scan_hist(needed):
        def cond(st):
            b, cum = st
            return (cum < needed) & (b > 0)

        def step(st):
            b, cum = st
            b2 = b - 1
            return b2, cum + hist_at(b2)

        bstar, cum = lax.while_loop(
            cond, step, (np.int32(HIST), np.int32(0)))
        return bstar, needed - (cum - hist_at(bstar))

    def do_row(row):
        pltpu.sync_copy(in_hbm.at[row], xrow)

        clear_hist()

        @pl.loop(0, NV, unroll=8)
        def _(v):
            key = xrow[pl.ds(v * LANES, LANES)] & ABS_MASK
            plsc.addupdate_scatter(hist, [key >> 23], ones16)

        bstar, needed = scan_hist(np.int32(K))

        @pl.loop(0, NV, init_carry=np.int32(0), unroll=4)
        def nA(v, off):
            raw = xrow[pl.ds(v * LANES, LANES)]
            key = raw & ABS_MASK
            b = key >> 23
            meq = b == bstar
            plsc.store_compressed(av.at[pl.ds(off, LANES)], raw, mask=meq)
            plsc.store_compressed(
                ai.at[pl.ds(off, LANES)], v * LANES + iota, mask=meq)
            xrow[pl.ds(v * LANES, LANES)] = jnp.where(b > bstar, raw, 0)
            pc = plsc.all_reduce_population_count(meq)[0]
            return jnp.minimum(off + pc, CAP)

        def refine(sv, si, dv, di, n, needed, shift, nbits):
            bmask = np.int32((1 << nbits) - 1)
            trips = (n + LANES - 1) // LANES
            clear_hist()

            @pl.loop(0, trips)
            def _(v):
                m = (v * LANES + iota) < n
                key = sv[pl.ds(v * LANES, LANES)] & ABS_MASK
                plsc.addupdate_scatter(
                    hist, [(key >> shift) & bmask], ones16, mask=m)

            bstar, needed2 = scan_hist(needed)

            @pl.loop(0, trips, init_carry=np.int32(0))
            def n2(v, off):
                lanem = (v * LANES + iota) < n
                raw = sv[pl.ds(v * LANES, LANES)]
                idx = si[pl.ds(v * LANES, LANES)]
                b = ((raw & ABS_MASK) >> shift) & bmask
                mgt = lanem & (b > bstar)
                plsc.store_scatter(xrow, [idx], raw, mask=mgt)
                meq = lanem & (b == bstar)
                plsc.store_compressed(dv.at[pl.ds(off, LANES)], raw, mask=meq)
                plsc.store_compressed(di.at[pl.ds(off, LANES)], idx, mask=meq)
                pc = plsc.all_reduce_population_count(meq)[0]
                return jnp.minimum(off + pc, CAP)

            return n2, needed2

        nB, needed = refine(av, ai, bv, bi, nA, needed, 15, 8)
        nA2, needed = refine(bv, bi, av, ai, nB, needed, 7, 8)
        nT, needed = refine(av, ai, bv, bi, nA2, needed, 0, 7)

        tie_trips = (jnp.minimum(nT, needed) + LANES - 1) // LANES

        @pl.loop(0, tie_trips)
        def _(v):
            posv = v * LANES + iota
            m = (posv < nT) & (posv < needed)
            raw = bv[pl.ds(v * LANES, LANES)]
            idx = bi[pl.ds(v * LANES, LANES)]
            plsc.store_scatter(xrow, [idx], raw, mask=m)

        pltpu.sync_copy(xrow, out_hbm.at[row])

    @pl.loop(0, ROWS_PER_W)
    def _(r):
        do_row(wid * ROWS_PER_W + r)


@jax.jit
def kernel(input_):
    mesh = plsc.VectorSubcoreMesh(
        core_axis_name="c", subcore_axis_name="s",
        num_cores=NC, num_subcores=NS)
    f = pl.kernel(
        _body,
        out_type=jax.ShapeDtypeStruct((ROWS, COLS), jnp.int32),
        mesh=mesh,
        scratch_types=[
            pltpu.VMEM((COLS,), jnp.int32),
            pltpu.VMEM((HIST + LANES,), jnp.int32),
            pltpu.VMEM((CAPB,), jnp.int32),
            pltpu.VMEM((CAPB,), jnp.int32),
            pltpu.VMEM((CAPB,), jnp.int32),
            pltpu.VMEM((CAPB,), jnp.int32),
        ],
        compiler_params=pltpu.CompilerParams(needs_layout_passes=False),
        name="topk_abs_mask_sc",
    )
    bits = lax.bitcast_convert_type(input_, jnp.int32)
    return lax.bitcast_convert_type(f(bits), jnp.float32)
